# baseline (speedup 1.0000x reference)
import functools

import jax
import jax.numpy as jnp
from jax import lax
from jax.experimental import pallas as pl
from jax.experimental.pallas import tpu as pltpu
from jax.experimental.pallas import tpu_sc as plsc

NC = 2
NS = 16
NW = NC * NS
NBUF = 3
LW = 128


def _sc_gather(x128, idx, gw):
    e_tot = idx.shape[0]
    per_w = e_tot // NW
    ch = per_w // gw
    mesh = plsc.VectorSubcoreMesh(core_axis_name="c", subcore_axis_name="s")

    @functools.partial(
        pl.kernel,
        out_type=jax.ShapeDtypeStruct((e_tot, LW), x128.dtype),
        mesh=mesh,
        scratch_types=[
            pltpu.VMEM((per_w,), jnp.int32),
            pltpu.VMEM((NBUF, gw, LW), x128.dtype),
            pltpu.SemaphoreType.DMA((NBUF,)),
            pltpu.SemaphoreType.DMA((NBUF,)),
        ],
    )
    def k(x_hbm, i_hbm, o_hbm, idx_v, rows_v, gsems, osems):
        c = lax.axis_index("c")
        s = lax.axis_index("s")
        wid = s * NC + c
        base = wid * per_w
        pltpu.sync_copy(i_hbm.at[pl.ds(base, per_w)], idx_v)

        @pl.loop(0, ch - ch % NBUF, step=NBUF)
        def _(j):
            gcs = [pltpu.async_copy(
                x_hbm.at[idx_v.at[pl.ds((j + b) * gw, gw)]],
                rows_v.at[b], gsems.at[b]) for b in range(NBUF)]
            ocs = []
            for b in range(NBUF):
                gcs[b].wait()
                ocs.append(pltpu.async_copy(
                    rows_v.at[b],
                    o_hbm.at[pl.ds(base + (j + b) * gw, gw)],
                    osems.at[b]))
            for oc in ocs:
                oc.wait()

        done = ch - ch % NBUF
        for b in range(ch % NBUF):
            pltpu.async_copy(
                x_hbm.at[idx_v.at[pl.ds((done + b) * gw, gw)]],
                rows_v.at[b], gsems.at[b]).wait()
            pltpu.async_copy(rows_v.at[b],
                             o_hbm.at[pl.ds(base + (done + b) * gw, gw)],
                             osems.at[b]).wait()
        tail = per_w - ch * gw
        if tail:
            pltpu.async_copy(
                x_hbm.at[idx_v.at[pl.ds(ch * gw, tail)]],
                rows_v.at[0].at[pl.ds(0, tail)], gsems.at[0]).wait()
            pltpu.sync_copy(rows_v.at[0].at[pl.ds(0, tail)],
                            o_hbm.at[pl.ds(base + ch * gw, tail)])

    return k(x128, idx)


def _sc_scatter(msg, dst, zeros, n_pad, gw):
    e_tot = dst.shape[0]
    per_w = e_tot // NW
    ch = per_w // gw
    stripe = n_pad // NS
    mesh = plsc.VectorSubcoreMesh(core_axis_name="c", subcore_axis_name="s")

    @functools.partial(
        pl.kernel,
        out_type=jax.ShapeDtypeStruct((NC, n_pad, LW), jnp.float32),
        mesh=mesh,
        scratch_types=[
            pltpu.VMEM((per_w,), jnp.int32),
            pltpu.VMEM((NBUF, gw, LW), jnp.float32),
            pltpu.VMEM_SHARED((n_pad, LW), jnp.float32),
            pltpu.SemaphoreType.DMA((NBUF,)),
            pltpu.SemaphoreType.DMA((NBUF,)),
        ],
    )
    def k(m_hbm, i_hbm, z_hbm, o_hbm, idx_v, buf_v, agg_sh, lsems, ssems):
        c = lax.axis_index("c")
        s = lax.axis_index("s")
        wid = s * NC + c
        base = wid * per_w
        pltpu.sync_copy(z_hbm.at[pl.ds(s * stripe, stripe)],
                        agg_sh.at[pl.ds(s * stripe, stripe)])
        pltpu.sync_copy(i_hbm.at[pl.ds(base, per_w)], idx_v)
        plsc.subcore_barrier()

        @pl.loop(0, ch - ch % NBUF, step=NBUF)
        def _(j):
            lcs = [pltpu.async_copy(
                m_hbm.at[pl.ds(base + (j + b) * gw, gw)],
                buf_v.at[b], lsems.at[b]) for b in range(NBUF)]
            scs = []
            for b in range(NBUF):
                lcs[b].wait()
                scs.append(pltpu.async_copy(
                    buf_v.at[b], agg_sh.at[idx_v.at[pl.ds((j + b) * gw, gw)]],
                    ssems.at[b], add=True))
            for sc in scs:
                sc.wait()

        done = ch - ch % NBUF
        for b in range(ch % NBUF):
            pltpu.async_copy(m_hbm.at[pl.ds(base + (done + b) * gw, gw)],
                             buf_v.at[b], lsems.at[b]).wait()
            pltpu.async_copy(buf_v.at[b],
                             agg_sh.at[idx_v.at[pl.ds((done + b) * gw, gw)]],
                             ssems.at[b], add=True).wait()
        tail = per_w - ch * gw
        if tail:
            pltpu.async_copy(m_hbm.at[pl.ds(base + ch * gw, tail)],
                             buf_v.at[0].at[pl.ds(0, tail)],
                             lsems.at[0]).wait()
            pltpu.async_copy(buf_v.at[0].at[pl.ds(0, tail)],
                             agg_sh.at[idx_v.at[pl.ds(ch * gw, tail)]],
                             ssems.at[0], add=True).wait()

        plsc.subcore_barrier()
        pltpu.sync_copy(agg_sh.at[pl.ds(s * stripe, stripe)],
                        o_hbm.at[c, pl.ds(s * stripe, stripe)])

    return k(msg, dst, zeros)


def _msg_body(eat_ref, xs_ref, w1_ref, b1_ref, w2f_ref, rk_ref,
              cvec_ref, out_ref, *, kk, d_out):
    eat = eat_ref[...]
    h = jnp.maximum(
        lax.dot_general(eat, w1_ref[...], (((0,), (0,)), ((), ())),
                        preferred_element_type=jnp.float32)
        + b1_ref[...], 0.0)
    hb = h.astype(jnp.bfloat16)
    xsb = xs_ref[:, :48].astype(jnp.bfloat16)
    yb = jnp.dot(xsb, w2f_ref[...], preferred_element_type=jnp.float32)
    y = yb[:, :kk]
    bias_t = yb[:, kk:]
    hrep = jnp.dot(hb, rk_ref[...], preferred_element_type=jnp.float32)
    acc = y[:, 0:LW] * hrep[:, 0:LW]
    for j in range(1, kk // LW):
        acc = acc + y[:, j * LW:(j + 1) * LW] * hrep[:, j * LW:(j + 1) * LW]
    red = acc[:, 0:d_out]
    for a in range(1, LW // d_out):
        red = red + acc[:, a * d_out:(a + 1) * d_out]
    b128 = bias_t + cvec_ref[...]
    out_ref[...] = b128
    out_ref[:, 0:d_out] = b128[:, 0:d_out] + red


def _tc_messages(eat, xs, w1a, b1a, w2f, rk, cvec, be, kk, d_out, e_off):
    d_e = eat.shape[0]
    e_tot = xs.shape[0]
    grid = (e_tot // be,)
    off_b = e_off // be
    return pl.pallas_call(
        functools.partial(_msg_body, kk=kk, d_out=d_out),
        grid=grid,
        in_specs=[
            pl.BlockSpec((d_e, be), lambda i: (0, i + off_b)),
            pl.BlockSpec((be, LW), lambda i: (i, 0)),
            pl.BlockSpec(w1a.shape, lambda i: (0, 0)),
            pl.BlockSpec(b1a.shape, lambda i: (0, 0)),
            pl.BlockSpec(w2f.shape, lambda i: (0, 0)),
            pl.BlockSpec(rk.shape, lambda i: (0, 0)),
            pl.BlockSpec(cvec.shape, lambda i: (0, 0)),
        ],
        out_specs=pl.BlockSpec((be, LW), lambda i: (i, 0)),
        out_shape=jax.ShapeDtypeStruct((e_tot, LW), jnp.float32),
    )(eat, xs, w1a, b1a, w2f, rk, cvec)


def _fin_body(x_ref, pa_ref, pb_ref, pc_ref, pd_ref, wr_ref, b_ref, out_ref):
    p = (pa_ref[...] + pb_ref[...]) + (pc_ref[...] + pd_ref[...])
    cnt = jnp.maximum(p[:, 32:33], 1.0)
    agg = p[:, :32] / cnt
    out_ref[...] = (
        jnp.dot(x_ref[...], wr_ref[...], preferred_element_type=jnp.float32)
        + agg + b_ref[...])


def _tc_finalize(x128, parts, wrp, bias2, bn):
    n = x128.shape[0]
    pspec = pl.BlockSpec((bn, LW), lambda i: (i, 0))
    return pl.pallas_call(
        _fin_body,
        grid=(n // bn,),
        in_specs=[
            pl.BlockSpec((bn, LW), lambda i: (i, 0)),
            pspec, pspec, pspec, pspec,
            pl.BlockSpec(wrp.shape, lambda i: (0, 0)),
            pl.BlockSpec(bias2.shape, lambda i: (0, 0)),
        ],
        out_specs=pl.BlockSpec((bn, 32), lambda i: (i, 0)),
        out_shape=jax.ShapeDtypeStruct((n, 32), jnp.float32),
    )(x128, *parts, wrp, bias2)


def kernel(x, edge_index, edge_attr, W1, b1, W2, b2, W_root, bias):
    n, d_in = x.shape
    e_tot = edge_attr.shape[0]
    eh = W1.shape[1]
    d_out = W_root.shape[1]
    ip = 48
    kk = eh * d_out

    gw = 128
    n_pad = ((n + 8 * NS - 1) // (8 * NS)) * (8 * NS)

    src = edge_index[0]
    dst = edge_index[1]

    x128 = jnp.pad(x, ((0, n_pad - n), (0, LW - d_in)))
    w1a = jnp.pad(W1, ((0, 0), (0, ip - eh)))
    b1a = jnp.pad(b1, (0, ip - eh))[None, :]
    w2r = W2.reshape(eh, d_in, d_out)
    w2rp = jnp.pad(w2r, ((0, 0), (0, ip - d_in), (0, 0)))
    w2f = w2rp.transpose(1, 0, 2).reshape(ip, kk)
    b2r = jnp.pad(b2.reshape(d_in, d_out), ((0, ip - d_in), (0, LW - d_out)))
    w2fb = jnp.concatenate([w2f, b2r], axis=1).astype(jnp.bfloat16)
    rk = jnp.repeat(jnp.eye(eh, dtype=jnp.float32), d_out,
                    axis=1).astype(jnp.bfloat16)
    rk = jnp.pad(rk, ((0, ip - eh), (0, 0)))
    cvec = jnp.zeros((1, LW), jnp.float32).at[0, d_out].set(1.0)
    wrp = jnp.pad(W_root, ((0, LW - d_in), (0, 0)))
    bias2 = bias[None, :]
    zeros = jnp.zeros((n_pad, LW), jnp.float32)

    eat = edge_attr.T
    splits = [(0, e_tot * 3 // 5), (e_tot * 3 // 5, e_tot * 2 // 5)]
    parts = []
    xs_halves = [_sc_gather(x128, lax.slice(src, (off,), (off + ln,)), gw)
                 for off, ln in splits]
    for h, (off, ln) in enumerate(splits):
        msg = _tc_messages(eat, xs_halves[h], w1a, b1a, w2fb, rk, cvec,
                           be=3200, kk=kk, d_out=d_out, e_off=off)
        p = _sc_scatter(msg, lax.slice(dst, (off,), (off + ln,)),
                        zeros, n_pad, 64)
        parts.extend([p[0], p[1]])
    out = _tc_finalize(x128, parts, wrp, bias2, bn=n_pad // 8)
    return out[:n]


# speedup vs baseline: 1.0912x; 1.0912x over previous
import functools

import jax
import jax.numpy as jnp
from jax import lax
from jax.experimental import pallas as pl
from jax.experimental.pallas import tpu as pltpu
from jax.experimental.pallas import tpu_sc as plsc

NC = 2
NS = 16
NW = NC * NS
NBUF = 3
LW = 128


def _sc_gather(x128, idx, gw):
    e_tot = idx.shape[0]
    per_w = e_tot // NW
    ch = per_w // gw
    mesh = plsc.VectorSubcoreMesh(core_axis_name="c", subcore_axis_name="s")

    @functools.partial(
        pl.kernel,
        out_type=jax.ShapeDtypeStruct((e_tot, LW), x128.dtype),
        mesh=mesh,
        scratch_types=[
            pltpu.VMEM((per_w,), jnp.int32),
            pltpu.VMEM((NBUF, gw, LW), x128.dtype),
            pltpu.SemaphoreType.DMA((NBUF,)),
            pltpu.SemaphoreType.DMA((NBUF,)),
        ],
    )
    def k(x_hbm, i_hbm, o_hbm, idx_v, rows_v, gsems, osems):
        c = lax.axis_index("c")
        s = lax.axis_index("s")
        wid = s * NC + c
        base = wid * per_w
        pltpu.sync_copy(i_hbm.at[pl.ds(base, per_w)], idx_v)

        @pl.loop(0, ch - ch % NBUF, step=NBUF)
        def _(j):
            gcs = [pltpu.async_copy(
                x_hbm.at[idx_v.at[pl.ds((j + b) * gw, gw)]],
                rows_v.at[b], gsems.at[b]) for b in range(NBUF)]
            ocs = []
            for b in range(NBUF):
                gcs[b].wait()
                ocs.append(pltpu.async_copy(
                    rows_v.at[b],
                    o_hbm.at[pl.ds(base + (j + b) * gw, gw)],
                    osems.at[b]))
            for oc in ocs:
                oc.wait()

        done = ch - ch % NBUF
        for b in range(ch % NBUF):
            pltpu.async_copy(
                x_hbm.at[idx_v.at[pl.ds((done + b) * gw, gw)]],
                rows_v.at[b], gsems.at[b]).wait()
            pltpu.async_copy(rows_v.at[b],
                             o_hbm.at[pl.ds(base + (done + b) * gw, gw)],
                             osems.at[b]).wait()
        tail = per_w - ch * gw
        if tail:
            pltpu.async_copy(
                x_hbm.at[idx_v.at[pl.ds(ch * gw, tail)]],
                rows_v.at[0].at[pl.ds(0, tail)], gsems.at[0]).wait()
            pltpu.sync_copy(rows_v.at[0].at[pl.ds(0, tail)],
                            o_hbm.at[pl.ds(base + ch * gw, tail)])

    return k(x128, idx)


def _sc_scatter(msg, dst, init, n_pad, gw):
    e_tot = dst.shape[0]
    per_w = e_tot // NW
    ch = per_w // gw
    stripe = n_pad // NS
    mesh = plsc.VectorSubcoreMesh(core_axis_name="c", subcore_axis_name="s")

    @functools.partial(
        pl.kernel,
        out_type=jax.ShapeDtypeStruct((NC, n_pad, LW), jnp.float32),
        mesh=mesh,
        scratch_types=[
            pltpu.VMEM((per_w,), jnp.int32),
            pltpu.VMEM((NBUF, gw, LW), jnp.float32),
            pltpu.VMEM_SHARED((n_pad, LW), jnp.float32),
            pltpu.SemaphoreType.DMA((NBUF,)),
            pltpu.SemaphoreType.DMA((NBUF,)),
        ],
    )
    def k(m_hbm, i_hbm, z_hbm, o_hbm, idx_v, buf_v, agg_sh, lsems, ssems):
        c = lax.axis_index("c")
        s = lax.axis_index("s")
        wid = s * NC + c
        base = wid * per_w
        pltpu.sync_copy(z_hbm.at[c, pl.ds(s * stripe, stripe)],
                        agg_sh.at[pl.ds(s * stripe, stripe)])
        pltpu.sync_copy(i_hbm.at[pl.ds(base, per_w)], idx_v)
        plsc.subcore_barrier()

        @pl.loop(0, ch - ch % NBUF, step=NBUF)
        def _(j):
            lcs = [pltpu.async_copy(
                m_hbm.at[pl.ds(base + (j + b) * gw, gw)],
                buf_v.at[b], lsems.at[b]) for b in range(NBUF)]
            scs = []
            for b in range(NBUF):
                lcs[b].wait()
                scs.append(pltpu.async_copy(
                    buf_v.at[b], agg_sh.at[idx_v.at[pl.ds((j + b) * gw, gw)]],
                    ssems.at[b], add=True))
            for sc in scs:
                sc.wait()

        done = ch - ch % NBUF
        for b in range(ch % NBUF):
            pltpu.async_copy(m_hbm.at[pl.ds(base + (done + b) * gw, gw)],
                             buf_v.at[b], lsems.at[b]).wait()
            pltpu.async_copy(buf_v.at[b],
                             agg_sh.at[idx_v.at[pl.ds((done + b) * gw, gw)]],
                             ssems.at[b], add=True).wait()
        tail = per_w - ch * gw
        if tail:
            pltpu.async_copy(m_hbm.at[pl.ds(base + ch * gw, tail)],
                             buf_v.at[0].at[pl.ds(0, tail)],
                             lsems.at[0]).wait()
            pltpu.async_copy(buf_v.at[0].at[pl.ds(0, tail)],
                             agg_sh.at[idx_v.at[pl.ds(ch * gw, tail)]],
                             ssems.at[0], add=True).wait()

        plsc.subcore_barrier()
        pltpu.sync_copy(agg_sh.at[pl.ds(s * stripe, stripe)],
                        o_hbm.at[c, pl.ds(s * stripe, stripe)])

    return k(msg, dst, init)


def _msg_body(eat_ref, xs_ref, w1_ref, b1_ref, w2f_ref, rk_ref,
              cvec_ref, out_ref, *, kk, d_out):
    eat = eat_ref[...]
    h = jnp.maximum(
        lax.dot_general(eat, w1_ref[...], (((0,), (0,)), ((), ())),
                        preferred_element_type=jnp.float32)
        + b1_ref[...], 0.0)
    hb = h.astype(jnp.bfloat16)
    xsb = xs_ref[:, :48].astype(jnp.bfloat16)
    yb = jnp.dot(xsb, w2f_ref[...], preferred_element_type=jnp.float32)
    y = yb[:, :kk]
    bias_t = yb[:, kk:]
    hrep = jnp.dot(hb, rk_ref[...], preferred_element_type=jnp.float32)
    acc = y[:, 0:LW] * hrep[:, 0:LW]
    for j in range(1, kk // LW):
        acc = acc + y[:, j * LW:(j + 1) * LW] * hrep[:, j * LW:(j + 1) * LW]
    red = acc[:, 0:d_out]
    for a in range(1, LW // d_out):
        red = red + acc[:, a * d_out:(a + 1) * d_out]
    b128 = bias_t + cvec_ref[...]
    out_ref[...] = b128
    out_ref[:, 0:d_out] = b128[:, 0:d_out] + red


def _tc_messages(eat, xs, w1a, b1a, w2f, rk, cvec, be, kk, d_out, e_off):
    d_e = eat.shape[0]
    e_tot = xs.shape[0]
    grid = (e_tot // be,)
    off_b = e_off // be
    return pl.pallas_call(
        functools.partial(_msg_body, kk=kk, d_out=d_out),
        grid=grid,
        in_specs=[
            pl.BlockSpec((d_e, be), lambda i: (0, i + off_b)),
            pl.BlockSpec((be, LW), lambda i: (i, 0)),
            pl.BlockSpec(w1a.shape, lambda i: (0, 0)),
            pl.BlockSpec(b1a.shape, lambda i: (0, 0)),
            pl.BlockSpec(w2f.shape, lambda i: (0, 0)),
            pl.BlockSpec(rk.shape, lambda i: (0, 0)),
            pl.BlockSpec(cvec.shape, lambda i: (0, 0)),
        ],
        out_specs=pl.BlockSpec((be, LW), lambda i: (i, 0)),
        out_shape=jax.ShapeDtypeStruct((e_tot, LW), jnp.float32),
    )(eat, xs, w1a, b1a, w2f, rk, cvec)


def _fin_body(x_ref, pa_ref, pb_ref, wr_ref, b_ref, out_ref):
    p = pa_ref[0] + pb_ref[0]
    cnt = jnp.maximum(p[:, 32:33], 1.0)
    agg = p[:, :32] / cnt
    out_ref[...] = (
        jnp.dot(x_ref[...], wr_ref[...], preferred_element_type=jnp.float32)
        + agg + b_ref[...])


def _tc_finalize(x128, parts, wrp, bias2, bn):
    n = x128.shape[0]
    return pl.pallas_call(
        _fin_body,
        grid=(n // bn,),
        in_specs=[
            pl.BlockSpec((bn, LW), lambda i: (i, 0)),
            pl.BlockSpec((1, bn, LW), lambda i: (0, i, 0)),
            pl.BlockSpec((1, bn, LW), lambda i: (1, i, 0)),
            pl.BlockSpec(wrp.shape, lambda i: (0, 0)),
            pl.BlockSpec(bias2.shape, lambda i: (0, 0)),
        ],
        out_specs=pl.BlockSpec((bn, 32), lambda i: (i, 0)),
        out_shape=jax.ShapeDtypeStruct((n, 32), jnp.float32),
    )(x128, parts, parts, wrp, bias2)


def kernel(x, edge_index, edge_attr, W1, b1, W2, b2, W_root, bias):
    n, d_in = x.shape
    e_tot = edge_attr.shape[0]
    eh = W1.shape[1]
    d_out = W_root.shape[1]
    ip = 48
    kk = eh * d_out

    gw = 128
    n_pad = ((n + 8 * NS - 1) // (8 * NS)) * (8 * NS)

    src = edge_index[0]
    dst = edge_index[1]

    x128 = jnp.pad(x, ((0, n_pad - n), (0, LW - d_in)))
    w1a = jnp.pad(W1, ((0, 0), (0, ip - eh)))
    b1a = jnp.pad(b1, (0, ip - eh))[None, :]
    w2r = W2.reshape(eh, d_in, d_out)
    w2rp = jnp.pad(w2r, ((0, 0), (0, ip - d_in), (0, 0)))
    w2f = w2rp.transpose(1, 0, 2).reshape(ip, kk)
    b2r = jnp.pad(b2.reshape(d_in, d_out), ((0, ip - d_in), (0, LW - d_out)))
    w2fb = jnp.concatenate([w2f, b2r], axis=1).astype(jnp.bfloat16)
    rk = jnp.repeat(jnp.eye(eh, dtype=jnp.float32), d_out,
                    axis=1).astype(jnp.bfloat16)
    rk = jnp.pad(rk, ((0, ip - eh), (0, 0)))
    cvec = jnp.zeros((1, LW), jnp.float32).at[0, d_out].set(1.0)
    wrp = jnp.pad(W_root, ((0, LW - d_in), (0, 0)))
    bias2 = bias[None, :]
    zeros = jnp.zeros((NC, n_pad, LW), jnp.float32)

    eat = edge_attr.T
    be = 3200
    lens = [e_tot * 9 // 25, e_tot * 7 // 25, e_tot * 5 // 25, e_tot * 4 // 25]
    offs = [0, lens[0], lens[0] + lens[1], lens[0] + lens[1] + lens[2]]
    xs_h = [_sc_gather(x128, lax.slice(src, (o,), (o + ln,)), gw)
            for o, ln in zip(offs, lens)]
    acc = zeros
    for h, (o, ln) in enumerate(zip(offs, lens)):
        msg = _tc_messages(eat, xs_h[h], w1a, b1a, w2fb, rk, cvec,
                           be=be, kk=kk, d_out=d_out, e_off=o)
        acc = _sc_scatter(msg, lax.slice(dst, (o,), (o + ln,)),
                          acc, n_pad, 64)
    out = _tc_finalize(x128, acc, wrp, bias2, bn=n_pad // 8)
    return out[:n]
